# baseline (speedup 1.0000x reference)
import jax
import jax.numpy as jnp
from jax import lax
from jax.experimental import pallas as pl
from jax.experimental.pallas import tpu as pltpu
from jax.experimental.pallas import tpu_sc as plsc

N = 10000
E = 320000
D = 128
H = 128
B = 64
P = 16
NE = 8
L = 3

NC = 2
NS = 16
NW = NC * NS
CHUNK = 128
N_PAD = 10240
HALF = N_PAD // 2
CPW = 80
E_PAD = NW * CPW * CHUNK
EPW = E_PAD // NW
RPT = N_PAD // NS
RPT2 = HALF // NS
DEG_W = 16
NB = 4
PADQ = NB * CHUNK
LC = (EPW + PADQ) // CHUNK

_mesh_cache = []


def _mesh():
    if not _mesh_cache:
        _mesh_cache.append(plsc.VectorSubcoreMesh(
            core_axis_name="c", subcore_axis_name="s",
            num_cores=NC, num_subcores=NS))
    return _mesh_cache[0]


def _sc_deg_body(dst_hbm, zeros_hbm, out_hbm, dst_v, ones_v, acc_sp):
    c = lax.axis_index("c")
    s = lax.axis_index("s")
    w = c * NS + s
    pltpu.sync_copy(dst_hbm.at[w], dst_v)

    def _fill(i, _):
        ones_v[i, :] = jnp.ones((16,), jnp.float32)
        return 0

    lax.fori_loop(0, CHUNK, _fill, 0)

    row0 = s * RPT
    pltpu.sync_copy(zeros_hbm.at[pl.ds(row0, RPT)], acc_sp.at[pl.ds(row0, RPT)])
    plsc.subcore_barrier()

    def _step(j, _):
        pltpu.sync_copy(ones_v, acc_sp.at[dst_v.at[j]], add=True)
        return 0

    lax.fori_loop(0, CPW, _step, 0)
    plsc.subcore_barrier()
    pltpu.sync_copy(acc_sp.at[pl.ds(row0, RPT)],
                    out_hbm.at[c, pl.ds(row0, RPT)])


def _sc_deg(dst_p, zeros_deg):
    return pl.kernel(
        _sc_deg_body,
        out_type=jax.ShapeDtypeStruct((NC, N_PAD, DEG_W), jnp.float32),
        mesh=_mesh(),
        compiler_params=pltpu.CompilerParams(use_tc_tiling_on_sc=False),
        scratch_types=[
            pltpu.VMEM((CPW, CHUNK), jnp.int32),
            pltpu.VMEM((CHUNK, DEG_W), jnp.float32),
            pltpu.VMEM_SHARED((N_PAD, DEG_W), jnp.float32),
        ],
    )(dst_p, zeros_deg)


def _sc_part_body(src_hbm, dst_hbm, slist_hbm, dlist_hbm, cnt_hbm,
                  sstage, dstage, slist, dlist, cbuf):
    c = lax.axis_index("c")
    s = lax.axis_index("s")
    w = c * NS + s
    pltpu.sync_copy(src_hbm.at[w], sstage)
    pltpu.sync_copy(dst_hbm.at[w], dstage)

    lanes = lax.iota(jnp.int32, 16)

    def _bucket(half, sv, dv, off):
        if half == 0:
            m = dv < HALF
            dloc = dv
        else:
            m = dv >= HALF
            dloc = dv - HALF
        pos = plsc.cumsum(m.astype(jnp.int32))
        fidx = off + pos - 1
        q = lax.shift_right_logical(fidx, 7)
        r = jnp.bitwise_and(fidx, 127)
        plsc.store_scatter(slist.at[half], [q, r], sv, mask=m)
        plsc.store_scatter(dlist.at[half], [q, r], dloc, mask=m)
        return off + jnp.sum(m.astype(jnp.int32))

    def _it(i, carry):
        off0, off1 = carry
        sv = sstage[pl.ds(i * 16, 16)]
        dv = dstage[pl.ds(i * 16, 16)]
        off0 = _bucket(0, sv, dv, off0)
        off1 = _bucket(1, sv, dv, off1)
        return off0, off1

    zero = jnp.zeros((), jnp.int32)
    off0, off1 = lax.fori_loop(0, EPW // 16, _it, (zero, zero))

    sent_s = jnp.full((16,), N_PAD - 1, jnp.int32)
    sent_d = jnp.zeros((16,), jnp.int32)
    for half, off in ((0, off0), (1, off1)):
        for k in range(PADQ // 16):
            fidx = off + k * 16 + lanes
            q = lax.shift_right_logical(fidx, 7)
            r = jnp.bitwise_and(fidx, 127)
            plsc.store_scatter(slist.at[half], [q, r], sent_s)
            plsc.store_scatter(dlist.at[half], [q, r], sent_d)
    p0 = jnp.bitwise_and(off0 + PADQ, -PADQ)
    p1 = jnp.bitwise_and(off1 + PADQ, -PADQ)
    cbuf[...] = jnp.where(lanes == 0, p0, jnp.where(lanes == 1, p1, 0))

    pltpu.sync_copy(slist, slist_hbm.at[w])
    pltpu.sync_copy(dlist, dlist_hbm.at[w])
    pltpu.sync_copy(cbuf, cnt_hbm.at[w])


def _sc_part(src_w, dst_w):
    return pl.kernel(
        _sc_part_body,
        out_type=(
            jax.ShapeDtypeStruct((NW, NC, LC, CHUNK), jnp.int32),
            jax.ShapeDtypeStruct((NW, NC, LC, CHUNK), jnp.int32),
            jax.ShapeDtypeStruct((NW, 16), jnp.int32),
        ),
        mesh=_mesh(),
        compiler_params=pltpu.CompilerParams(use_tc_tiling_on_sc=False,
                                             needs_layout_passes=False),
        scratch_types=[
            pltpu.VMEM((EPW,), jnp.int32),
            pltpu.VMEM((EPW,), jnp.int32),
            pltpu.VMEM((NC, LC, CHUNK), jnp.int32),
            pltpu.VMEM((NC, LC, CHUNK), jnp.int32),
            pltpu.VMEM((16,), jnp.int32),
        ],
    )(src_w, dst_w)


def _sc_scatter_body(g_hbm, slist_hbm, dlist_hbm, cnt_hbm, zeros_hbm, out_hbm,
                     cnts_v, sidx, didx, rows, *sems):
    gsems = sems[:NB]
    ssems = sems[NB:2 * NB]
    isem = sems[2 * NB]
    acc_sp = sems[2 * NB + 1]
    c = lax.axis_index("c")
    s = lax.axis_index("s")

    row0 = s * RPT2
    pltpu.sync_copy(zeros_hbm.at[pl.ds(row0, RPT2)],
                    acc_sp.at[pl.ds(row0, RPT2)])
    pltpu.sync_copy(cnt_hbm, cnts_v)
    plsc.subcore_barrier()

    def _drain(w):
        cv = cnts_v[w, :]
        n = jnp.where(c == 0, cv[0], cv[1])
        nblk = lax.shift_right_logical(n, 9)
        sl = slist_hbm.at[w, c]
        dl = dlist_hbm.at[w, c]

        def _idx_start(t, p):
            pltpu.async_copy(sl.at[pl.ds(t * NB, NB)], sidx.at[p], isem)
            pltpu.async_copy(dl.at[pl.ds(t * NB, NB)], didx.at[p], isem)

        def _idx_wait(t, p):
            pltpu.make_async_copy(sl.at[pl.ds(t * NB, NB)], sidx.at[p],
                                  isem).wait()
            pltpu.make_async_copy(dl.at[pl.ds(t * NB, NB)], didx.at[p],
                                  isem).wait()

        def _gather(p, b):
            pltpu.async_copy(g_hbm.at[sidx.at[p, b]], rows.at[b], gsems[b])

        def _gwait(p, b):
            pltpu.make_async_copy(g_hbm.at[sidx.at[p, b]], rows.at[b],
                                  gsems[b]).wait()

        def _scatter(p, b):
            pltpu.async_copy(rows.at[b], acc_sp.at[didx.at[p, b]], ssems[b],
                             add=True)

        def _swait(p, b):
            pltpu.make_async_copy(rows.at[b], acc_sp.at[didx.at[p, b]],
                                  ssems[b]).wait()

        _idx_start(0, 0)
        _idx_wait(0, 0)
        for b in range(NB):
            _gather(0, b)

        @pl.when(1 < nblk)
        def _():
            _idx_start(1, 1)

        def _block(t, _):
            p = lax.rem(t, 2)
            pn = 1 - p
            for b in range(NB):
                _gwait(p, b)
                _scatter(p, b)

            @pl.when(t + 1 < nblk)
            def _():
                _idx_wait(t + 1, pn)

            for b in range(NB):
                _swait(p, b)

                @pl.when(t + 1 < nblk)
                def _():
                    _gather(pn, b)

            @pl.when(t + 2 < nblk)
            def _():
                _idx_start(t + 2, p)

            return 0

        lax.fori_loop(0, nblk, _block, 0)

    _drain(s * 2)
    _drain(s * 2 + 1)

    plsc.subcore_barrier()
    pltpu.sync_copy(acc_sp.at[pl.ds(row0, RPT2)],
                    out_hbm.at[pl.ds(c * HALF + row0, RPT2)])


def _sc_scatter(g, slist, dlist, cnts, zeros_half):
    return pl.kernel(
        _sc_scatter_body,
        out_type=jax.ShapeDtypeStruct((N_PAD, H), jnp.float32),
        mesh=_mesh(),
        compiler_params=pltpu.CompilerParams(use_tc_tiling_on_sc=False),
        scratch_types=(
            [pltpu.VMEM((NW, 16), jnp.int32),
             pltpu.VMEM((2, NB, CHUNK), jnp.int32),
             pltpu.VMEM((2, NB, CHUNK), jnp.int32),
             pltpu.VMEM((NB, CHUNK, H), jnp.float32)]
            + [pltpu.SemaphoreType.DMA] * (2 * NB + 1)
            + [pltpu.VMEM_SHARED((HALF, H), jnp.float32)]
        ),
    )(g, slist, dlist, cnts, zeros_half)


BLK = 1024
GRID = N_PAD // BLK


def _dinv_block(degp_ref):
    deg = degp_ref[0, :, 0:1] + degp_ref[1, :, 0:1] + 1.0
    return lax.rsqrt(deg)


def _row_mask(i):
    rid = lax.broadcasted_iota(jnp.int32, (BLK, 1), 0) + i * BLK
    return rid < N


def _tc_first_body(x_ref, w_ref, degp_ref, g_ref):
    i = pl.program_id(0)
    dinv = _dinv_block(degp_ref)
    xw = jnp.dot(x_ref[...], w_ref[...], preferred_element_type=jnp.float32)
    g_ref[...] = jnp.where(_row_mask(i), xw * dinv, 0.0)


def _tc_first(x_pad, w, degp):
    return pl.pallas_call(
        _tc_first_body,
        grid=(GRID,),
        in_specs=[
            pl.BlockSpec((BLK, D), lambda i: (i, 0)),
            pl.BlockSpec((D, H), lambda i: (0, 0)),
            pl.BlockSpec((NC, BLK, DEG_W), lambda i: (0, i, 0)),
        ],
        out_specs=pl.BlockSpec((BLK, H), lambda i: (i, 0)),
        out_shape=jax.ShapeDtypeStruct((N_PAD, H), jnp.float32),
    )(x_pad, w, degp)


def _tc_combine_body(acc_ref, g_ref, degp_ref, b_ref, w_ref, gout_ref):
    i = pl.program_id(0)
    dinv = _dinv_block(degp_ref)
    h = jnp.maximum((acc_ref[...] + g_ref[...]) * dinv + b_ref[...], 0.0)
    gw = jnp.dot(h, w_ref[...], preferred_element_type=jnp.float32)
    gout_ref[...] = jnp.where(_row_mask(i), gw * dinv, 0.0)


def _tc_combine(acc, g, degp, b_row, w):
    return pl.pallas_call(
        _tc_combine_body,
        grid=(GRID,),
        in_specs=[
            pl.BlockSpec((BLK, H), lambda i: (i, 0)),
            pl.BlockSpec((BLK, H), lambda i: (i, 0)),
            pl.BlockSpec((NC, BLK, DEG_W), lambda i: (0, i, 0)),
            pl.BlockSpec((1, H), lambda i: (0, 0)),
            pl.BlockSpec((H, H), lambda i: (0, 0)),
        ],
        out_specs=pl.BlockSpec((BLK, H), lambda i: (i, 0)),
        out_shape=jax.ShapeDtypeStruct((N_PAD, H), jnp.float32),
    )(acc, g, degp, b_row, w)


def _tc_pool_body(acc_ref, g_ref, degp_ref, b_ref, batch_ref,
                  sums_ref, counts_ref):
    i = pl.program_id(0)
    dinv = _dinv_block(degp_ref)
    h = jnp.maximum((acc_ref[...] + g_ref[...]) * dinv + b_ref[...], 0.0)
    bt = batch_ref[...]
    onehot = (bt == lax.broadcasted_iota(jnp.int32, (BLK, B), 1)
              ).astype(jnp.float32)
    dn = (((0,), (0,)), ((), ()))
    part = lax.dot_general(onehot, h, dn, preferred_element_type=jnp.float32)
    cnt = lax.dot_general(onehot, jnp.ones((BLK, 8), jnp.float32), dn,
                          preferred_element_type=jnp.float32)

    @pl.when(i == 0)
    def _():
        sums_ref[...] = jnp.zeros_like(sums_ref)
        counts_ref[...] = jnp.zeros_like(counts_ref)

    sums_ref[...] += part
    counts_ref[...] += cnt


def _tc_pool(acc, g, degp, b_row, batch_pad):
    return pl.pallas_call(
        _tc_pool_body,
        grid=(GRID,),
        in_specs=[
            pl.BlockSpec((BLK, H), lambda i: (i, 0)),
            pl.BlockSpec((BLK, H), lambda i: (i, 0)),
            pl.BlockSpec((NC, BLK, DEG_W), lambda i: (0, i, 0)),
            pl.BlockSpec((1, H), lambda i: (0, 0)),
            pl.BlockSpec((BLK, 1), lambda i: (i, 0)),
        ],
        out_specs=[
            pl.BlockSpec((B, H), lambda i: (0, 0)),
            pl.BlockSpec((B, 8), lambda i: (0, 0)),
        ],
        out_shape=[
            jax.ShapeDtypeStruct((B, H), jnp.float32),
            jax.ShapeDtypeStruct((B, 8), jnp.float32),
        ],
    )(acc, g, degp, b_row, batch_pad)


def _tc_head_body(sums_ref, counts_ref, gw_ref, gb_ref, e1_ref, w11_ref,
                  b11_ref, w12_ref, b12_ref, in2_ref, w21_ref, b21_ref,
                  w22_ref, b22_ref, f1w_ref, f1b_ref, f2w_ref, out_ref):
    gx = sums_ref[...] / jnp.maximum(counts_ref[:, 0:1], 1.0)
    gx = jnp.maximum(
        jnp.dot(gx, gw_ref[...], preferred_element_type=jnp.float32)
        + gb_ref[...], 0.0)
    e = jnp.maximum(
        jnp.dot(e1_ref[...], w11_ref[...], preferred_element_type=jnp.float32)
        + b11_ref[...], 0.0)
    e = jnp.maximum(
        jnp.dot(e, w12_ref[...], preferred_element_type=jnp.float32)
        + b12_ref[...], 0.0)
    pool = jnp.where(
        lax.broadcasted_iota(jnp.int32, (B, B * NE), 1) // NE
        == lax.broadcasted_iota(jnp.int32, (B, B * NE), 0),
        1.0 / NE, 0.0)
    i1 = jnp.dot(pool, e, preferred_element_type=jnp.float32)
    i2 = jnp.maximum(
        jnp.dot(in2_ref[...], w21_ref[...], preferred_element_type=jnp.float32)
        + b21_ref[...], 0.0)
    i2 = jnp.maximum(
        jnp.dot(i2, w22_ref[...], preferred_element_type=jnp.float32)
        + b22_ref[...], 0.0)
    o = jnp.maximum(
        jnp.dot(gx, f1w_ref[0:8], preferred_element_type=jnp.float32)
        + jnp.dot(i1, f1w_ref[8:16], preferred_element_type=jnp.float32)
        + jnp.dot(i2, f1w_ref[16:24], preferred_element_type=jnp.float32)
        + f1b_ref[...], 0.0)
    out_ref[...] = jnp.dot(o, f2w_ref[...], preferred_element_type=jnp.float32)


def _tc_head(sums, counts, gw, gb_row, e1, w11, b11_row, w12, b12_row,
             in2_p, w21_p, b21_row, w22, b22_row, f1w, f1b_row, f2w_p):
    return pl.pallas_call(
        _tc_head_body,
        out_shape=jax.ShapeDtypeStruct((B, 8), jnp.float32),
    )(sums, counts, gw, gb_row, e1, w11, b11_row, w12, b12_row,
      in2_p, w21_p, b21_row, w22, b22_row, f1w, f1b_row, f2w_p)


def kernel(x, edge_index, batch, input1, input2, conv_W, conv_b,
           graph_fc_W, graph_fc_b, in1_fc1_W, in1_fc1_b, in1_fc2_W,
           in1_fc2_b, in2_fc1_W, in2_fc1_b, in2_fc2_W, in2_fc2_b,
           final1_W, final1_b, final2_W, final2_b):
    f32 = jnp.float32
    pad_e = E_PAD - E
    fillv = jnp.full((pad_e,), N_PAD - 1, jnp.int32)
    src_flat = jnp.concatenate([edge_index[0].astype(jnp.int32), fillv])
    dst_flat = jnp.concatenate([edge_index[1].astype(jnp.int32), fillv])
    src_w = src_flat.reshape(NW, EPW)
    dst_w = dst_flat.reshape(NW, EPW)
    dst_c = dst_flat.reshape(NW, CPW, CHUNK)
    x_pad = jnp.pad(x, ((0, N_PAD - N), (0, 0)))
    batch_pad = jnp.concatenate(
        [batch.astype(jnp.int32), jnp.full((N_PAD - N,), B, jnp.int32)]
    ).reshape(N_PAD, 1)
    zeros_deg = jnp.zeros((N_PAD, DEG_W), f32)
    zeros_half = jnp.zeros((HALF, H), f32)

    degp = _sc_deg(dst_c, zeros_deg)
    slist, dlist, cnts = _sc_part(src_w, dst_w)

    g = _tc_first(x_pad, conv_W[0], degp)
    for l in range(L - 1):
        acc = _sc_scatter(g, slist, dlist, cnts, zeros_half)
        g = _tc_combine(acc, g, degp, conv_b[l].reshape(1, H), conv_W[l + 1])
    acc = _sc_scatter(g, slist, dlist, cnts, zeros_half)
    sums, counts = _tc_pool(acc, g, degp, conv_b[L - 1].reshape(1, H),
                            batch_pad)

    e1 = input1.reshape(B * NE, P)
    in2_p = jnp.pad(input2, ((0, 0), (0, 6)))
    w21_p = jnp.pad(in2_fc1_W, ((0, 6), (0, 0)))
    f2w_p = jnp.pad(final2_W, ((0, 4), (0, 7)))
    f1w_p = jnp.pad(final1_W, ((0, 0), (0, 4)))
    f1b_p = jnp.pad(final1_b, (0, 4)).reshape(1, 16)

    out = _tc_head(sums, counts, graph_fc_W, graph_fc_b.reshape(1, 8),
                   e1, in1_fc1_W, in1_fc1_b.reshape(1, H),
                   in1_fc2_W, in1_fc2_b.reshape(1, 8),
                   in2_p, w21_p, in2_fc1_b.reshape(1, H),
                   in2_fc2_W, in2_fc2_b.reshape(1, 8),
                   f1w_p, f1b_p, f2w_p)
    return out[:, 0:1] + final2_b


# speedup vs baseline: 3.0077x; 3.0077x over previous
import jax
import jax.numpy as jnp
from jax import lax
from jax.experimental import pallas as pl
from jax.experimental.pallas import tpu as pltpu
from jax.experimental.pallas import tpu_sc as plsc

N = 10000
E = 320000
D = 128
H = 128
B = 64
P = 16
NE = 8
L = 3

NC = 2
NS = 16
NW = NC * NS
CHUNK = 128
N_PAD = 10240
CPW = 80
E_PAD = NW * CPW * CHUNK
CPS = E_PAD // (NS * CHUNK)
RPT = N_PAD // NS
DEG_W = 16
DH = D // 2

_mesh_cache = []


def _mesh():
    if not _mesh_cache:
        _mesh_cache.append(plsc.VectorSubcoreMesh(
            core_axis_name="c", subcore_axis_name="s",
            num_cores=NC, num_subcores=NS))
    return _mesh_cache[0]


def _sc_deg_body(dst_hbm, zeros_hbm, out_hbm, dst_v, ones_v, acc_sp):
    c = lax.axis_index("c")
    s = lax.axis_index("s")
    w = c * NS + s
    pltpu.sync_copy(dst_hbm.at[w], dst_v)

    def _fill(i, _):
        ones_v[i, :] = jnp.ones((16,), jnp.float32)
        return 0

    lax.fori_loop(0, CHUNK, _fill, 0)

    row0 = s * RPT
    pltpu.sync_copy(zeros_hbm.at[pl.ds(row0, RPT)], acc_sp.at[pl.ds(row0, RPT)])
    plsc.subcore_barrier()

    def _step(j, _):
        pltpu.sync_copy(ones_v, acc_sp.at[dst_v.at[j]], add=True)
        return 0

    lax.fori_loop(0, CPW, _step, 0)
    plsc.subcore_barrier()
    pltpu.sync_copy(acc_sp.at[pl.ds(row0, RPT)],
                    out_hbm.at[c, pl.ds(row0, RPT)])


def _sc_deg(dst_p, zeros_deg):
    return pl.kernel(
        _sc_deg_body,
        out_type=jax.ShapeDtypeStruct((NC, N_PAD, DEG_W), jnp.float32),
        mesh=_mesh(),
        compiler_params=pltpu.CompilerParams(use_tc_tiling_on_sc=False),
        scratch_types=[
            pltpu.VMEM((CPW, CHUNK), jnp.int32),
            pltpu.VMEM((CHUNK, DEG_W), jnp.float32),
            pltpu.VMEM_SHARED((N_PAD, DEG_W), jnp.float32),
        ],
    )(dst_p, zeros_deg)


NB = 8
NBLK = CPS // NB


def _sc_scatter_body(g_hbm, src_hbm, dst_hbm, out_hbm,
                     sidx, didx, rows, *sems):
    gsems = sems[:NB]
    ssems = sems[NB:2 * NB]
    isem = sems[2 * NB]
    acc_sp = sems[2 * NB + 1]
    c = lax.axis_index("c")
    s = lax.axis_index("s")

    zbuf = rows.at[0]
    zv = jnp.zeros((16,), jnp.float32)

    def _zfill(i, _):
        for j in range(DH // 16):
            zbuf[i, pl.ds(j * 16, 16)] = zv
        return 0

    lax.fori_loop(0, CHUNK, _zfill, 0)
    row0 = s * RPT
    for t in range(RPT // CHUNK):
        pltpu.sync_copy(zbuf, acc_sp.at[pl.ds(row0 + t * CHUNK, CHUNK)])
    plsc.subcore_barrier()

    g_half = g_hbm.at[c]

    def _idx_start(t, p):
        pltpu.async_copy(src_hbm.at[s, pl.ds(t * NB, NB)], sidx.at[p], isem)
        pltpu.async_copy(dst_hbm.at[s, pl.ds(t * NB, NB)], didx.at[p], isem)

    def _idx_wait(t, p):
        pltpu.make_async_copy(src_hbm.at[s, pl.ds(t * NB, NB)], sidx.at[p],
                              isem).wait()
        pltpu.make_async_copy(dst_hbm.at[s, pl.ds(t * NB, NB)], didx.at[p],
                              isem).wait()

    def _gather(p, b):
        pltpu.async_copy(g_half.at[sidx.at[p, b]], rows.at[b], gsems[b])

    def _gwait(p, b):
        pltpu.make_async_copy(g_half.at[sidx.at[p, b]], rows.at[b],
                              gsems[b]).wait()

    def _scatter(p, b):
        pltpu.async_copy(rows.at[b], acc_sp.at[didx.at[p, b]], ssems[b],
                         add=True)

    def _swait(p, b):
        pltpu.make_async_copy(rows.at[b], acc_sp.at[didx.at[p, b]],
                              ssems[b]).wait()

    _idx_start(0, 0)
    _idx_wait(0, 0)
    for b in range(NB):
        _gather(0, b)
    _idx_start(1, 1)

    def _block(t, _):
        p = lax.rem(t, 2)
        pn = 1 - p
        for b in range(NB):
            _gwait(p, b)
            _scatter(p, b)
        @pl.when(t + 1 < NBLK)
        def _():
            _idx_wait(t + 1, pn)

        for b in range(NB):
            _swait(p, b)

            @pl.when(t + 1 < NBLK)
            def _():
                _gather(pn, b)

        @pl.when(t + 2 < NBLK)
        def _():
            _idx_start(t + 2, p)

        return 0

    lax.fori_loop(0, NBLK, _block, 0)
    plsc.subcore_barrier()
    pltpu.sync_copy(acc_sp.at[pl.ds(row0, RPT)],
                    out_hbm.at[c, pl.ds(row0, RPT)])


def _sc_scatter(g, src_p, dst_p):
    return pl.kernel(
        _sc_scatter_body,
        out_type=jax.ShapeDtypeStruct((NC, N_PAD, DH), jnp.float32),
        mesh=_mesh(),
        compiler_params=pltpu.CompilerParams(use_tc_tiling_on_sc=False),
        scratch_types=(
            [pltpu.VMEM((2, NB, CHUNK), jnp.int32),
             pltpu.VMEM((2, NB, CHUNK), jnp.int32),
             pltpu.VMEM((NB, CHUNK, DH), jnp.float32)]
            + [pltpu.SemaphoreType.DMA] * (2 * NB + 1)
            + [pltpu.VMEM_SHARED((N_PAD, DH), jnp.float32)]
        ),
    )(g, src_p, dst_p)


BLK = 1024
GRID = N_PAD // BLK


def _dinv_block(degp_ref):
    deg = degp_ref[0, :, 0:1] + degp_ref[1, :, 0:1] + 1.0
    return lax.rsqrt(deg)


def _tc_first_body(x_ref, w_ref, degp_ref, g_ref):
    dinv = _dinv_block(degp_ref)
    x = x_ref[...]
    g_ref[0] = jnp.dot(x, w_ref[0], preferred_element_type=jnp.float32) * dinv
    g_ref[1] = jnp.dot(x, w_ref[1], preferred_element_type=jnp.float32) * dinv


def _tc_first(x_pad, w2, degp):
    return pl.pallas_call(
        _tc_first_body,
        grid=(GRID,),
        in_specs=[
            pl.BlockSpec((BLK, D), lambda i: (i, 0)),
            pl.BlockSpec((NC, D, DH), lambda i: (0, 0, 0)),
            pl.BlockSpec((NC, BLK, DEG_W), lambda i: (0, i, 0)),
        ],
        out_specs=pl.BlockSpec((NC, BLK, DH), lambda i: (0, i, 0)),
        out_shape=jax.ShapeDtypeStruct((NC, N_PAD, DH), jnp.float32),
    )(x_pad, w2, degp)


def _halves(acc_ref, g_ref, degp_ref, b_ref):
    dinv = _dinv_block(degp_ref)
    h0 = jnp.maximum((acc_ref[0] + g_ref[0]) * dinv + b_ref[0], 0.0)
    h1 = jnp.maximum((acc_ref[1] + g_ref[1]) * dinv + b_ref[1], 0.0)
    return dinv, h0, h1


def _tc_combine_body(acc_ref, g_ref, degp_ref, b_ref, w_ref, gout_ref):
    dinv, h0, h1 = _halves(acc_ref, g_ref, degp_ref, b_ref)
    for m in range(NC):
        gout_ref[m] = (
            jnp.dot(h0, w_ref[0, m], preferred_element_type=jnp.float32)
            + jnp.dot(h1, w_ref[1, m], preferred_element_type=jnp.float32)
        ) * dinv


def _tc_combine(acc, g, degp, b2, w4):
    return pl.pallas_call(
        _tc_combine_body,
        grid=(GRID,),
        in_specs=[
            pl.BlockSpec((NC, BLK, DH), lambda i: (0, i, 0)),
            pl.BlockSpec((NC, BLK, DH), lambda i: (0, i, 0)),
            pl.BlockSpec((NC, BLK, DEG_W), lambda i: (0, i, 0)),
            pl.BlockSpec((NC, 1, DH), lambda i: (0, 0, 0)),
            pl.BlockSpec((NC, NC, DH, DH), lambda i: (0, 0, 0, 0)),
        ],
        out_specs=pl.BlockSpec((NC, BLK, DH), lambda i: (0, i, 0)),
        out_shape=jax.ShapeDtypeStruct((NC, N_PAD, DH), jnp.float32),
    )(acc, g, degp, b2, w4)


def _tc_pool_body(acc_ref, g_ref, degp_ref, b_ref, batch_ref,
                  sums_ref, counts_ref):
    i = pl.program_id(0)
    _, h0, h1 = _halves(acc_ref, g_ref, degp_ref, b_ref)
    bt = batch_ref[...]
    onehot = (bt == lax.broadcasted_iota(jnp.int32, (BLK, B), 1)
              ).astype(jnp.float32)
    dn = (((0,), (0,)), ((), ()))
    part0 = lax.dot_general(onehot, h0, dn, preferred_element_type=jnp.float32)
    part1 = lax.dot_general(onehot, h1, dn, preferred_element_type=jnp.float32)
    cnt = lax.dot_general(onehot, jnp.ones((BLK, 8), jnp.float32), dn,
                          preferred_element_type=jnp.float32)

    @pl.when(i == 0)
    def _():
        sums_ref[...] = jnp.zeros_like(sums_ref)
        counts_ref[...] = jnp.zeros_like(counts_ref)

    sums_ref[0] += part0
    sums_ref[1] += part1
    counts_ref[...] += cnt


def _tc_pool(acc, g, degp, b2, batch_pad):
    return pl.pallas_call(
        _tc_pool_body,
        grid=(GRID,),
        in_specs=[
            pl.BlockSpec((NC, BLK, DH), lambda i: (0, i, 0)),
            pl.BlockSpec((NC, BLK, DH), lambda i: (0, i, 0)),
            pl.BlockSpec((NC, BLK, DEG_W), lambda i: (0, i, 0)),
            pl.BlockSpec((NC, 1, DH), lambda i: (0, 0, 0)),
            pl.BlockSpec((BLK, 1), lambda i: (i, 0)),
        ],
        out_specs=[
            pl.BlockSpec((NC, B, DH), lambda i: (0, 0, 0)),
            pl.BlockSpec((B, 8), lambda i: (0, 0)),
        ],
        out_shape=[
            jax.ShapeDtypeStruct((NC, B, DH), jnp.float32),
            jax.ShapeDtypeStruct((B, 8), jnp.float32),
        ],
    )(acc, g, degp, b2, batch_pad)


def _tc_head_body(sums_ref, counts_ref, gw_ref, gb_ref, e1_ref, w11_ref,
                  b11_ref, w12_ref, b12_ref, in2_ref, w21_ref, b21_ref,
                  w22_ref, b22_ref, f1w_ref, f1b_ref, f2w_ref, out_ref):
    icnt = 1.0 / jnp.maximum(counts_ref[:, 0:1], 1.0)
    gx0 = sums_ref[0] * icnt
    gx1 = sums_ref[1] * icnt
    gx = jnp.maximum(
        jnp.dot(gx0, gw_ref[0:DH], preferred_element_type=jnp.float32)
        + jnp.dot(gx1, gw_ref[DH:D], preferred_element_type=jnp.float32)
        + gb_ref[...], 0.0)
    e = jnp.maximum(
        jnp.dot(e1_ref[...], w11_ref[...], preferred_element_type=jnp.float32)
        + b11_ref[...], 0.0)
    e = jnp.maximum(
        jnp.dot(e, w12_ref[...], preferred_element_type=jnp.float32)
        + b12_ref[...], 0.0)
    pool = jnp.where(
        lax.broadcasted_iota(jnp.int32, (B, B * NE), 1) // NE
        == lax.broadcasted_iota(jnp.int32, (B, B * NE), 0),
        1.0 / NE, 0.0)
    i1 = jnp.dot(pool, e, preferred_element_type=jnp.float32)
    i2 = jnp.maximum(
        jnp.dot(in2_ref[...], w21_ref[...], preferred_element_type=jnp.float32)
        + b21_ref[...], 0.0)
    i2 = jnp.maximum(
        jnp.dot(i2, w22_ref[...], preferred_element_type=jnp.float32)
        + b22_ref[...], 0.0)
    o = jnp.maximum(
        jnp.dot(gx, f1w_ref[0:8], preferred_element_type=jnp.float32)
        + jnp.dot(i1, f1w_ref[8:16], preferred_element_type=jnp.float32)
        + jnp.dot(i2, f1w_ref[16:24], preferred_element_type=jnp.float32)
        + f1b_ref[...], 0.0)
    out_ref[...] = jnp.dot(o, f2w_ref[...], preferred_element_type=jnp.float32)


def _tc_head(sums, counts, gw, gb_row, e1, w11, b11_row, w12, b12_row,
             in2_p, w21_p, b21_row, w22, b22_row, f1w, f1b_row, f2w_p):
    return pl.pallas_call(
        _tc_head_body,
        out_shape=jax.ShapeDtypeStruct((B, 8), jnp.float32),
    )(sums, counts, gw, gb_row, e1, w11, b11_row, w12, b12_row,
      in2_p, w21_p, b21_row, w22, b22_row, f1w, f1b_row, f2w_p)


def _split_w(w):
    return w.reshape(NC, DH, NC, DH).transpose(0, 2, 1, 3)


def kernel(x, edge_index, batch, input1, input2, conv_W, conv_b,
           graph_fc_W, graph_fc_b, in1_fc1_W, in1_fc1_b, in1_fc2_W,
           in1_fc2_b, in2_fc1_W, in2_fc1_b, in2_fc2_W, in2_fc2_b,
           final1_W, final1_b, final2_W, final2_b):
    f32 = jnp.float32
    pad_e = E_PAD - E
    fillv = jnp.full((pad_e,), N_PAD - 1, jnp.int32)
    src_flat = jnp.concatenate([edge_index[0].astype(jnp.int32), fillv])
    dst_flat = jnp.concatenate([edge_index[1].astype(jnp.int32), fillv])
    src_s = src_flat.reshape(NS, CPS, CHUNK)
    dst_s = dst_flat.reshape(NS, CPS, CHUNK)
    dst_w = dst_flat.reshape(NW, CPW, CHUNK)
    x_pad = jnp.pad(x, ((0, N_PAD - N), (0, 0)))
    batch_pad = jnp.concatenate(
        [batch.astype(jnp.int32), jnp.full((N_PAD - N,), B, jnp.int32)]
    ).reshape(N_PAD, 1)
    zeros_deg = jnp.zeros((N_PAD, DEG_W), f32)

    degp = _sc_deg(dst_w, zeros_deg)

    w0 = jnp.stack([conv_W[0][:, :DH], conv_W[0][:, DH:]])
    g = _tc_first(x_pad, w0, degp)
    for l in range(L - 1):
        acc = _sc_scatter(g, src_s, dst_s)
        b2 = conv_b[l].reshape(NC, 1, DH)
        g = _tc_combine(acc, g, degp, b2, _split_w(conv_W[l + 1]))
    acc = _sc_scatter(g, src_s, dst_s)
    sums, counts = _tc_pool(acc, g, degp, conv_b[L - 1].reshape(NC, 1, DH),
                            batch_pad)

    e1 = input1.reshape(B * NE, P)
    in2_p = jnp.pad(input2, ((0, 0), (0, 6)))
    w21_p = jnp.pad(in2_fc1_W, ((0, 6), (0, 0)))
    f2w_p = jnp.pad(final2_W, ((0, 4), (0, 7)))
    f1w_p = jnp.pad(final1_W, ((0, 0), (0, 4)))
    f1b_p = jnp.pad(final1_b, (0, 4)).reshape(1, 16)

    out = _tc_head(sums, counts, graph_fc_W, graph_fc_b.reshape(1, 8),
                   e1, in1_fc1_W, in1_fc1_b.reshape(1, H),
                   in1_fc2_W, in1_fc2_b.reshape(1, 8),
                   in2_p, w21_p, in2_fc1_b.reshape(1, H),
                   in2_fc2_W, in2_fc2_b.reshape(1, 8),
                   f1w_p, f1b_p, f2w_p)
    return out[:, 0:1] + final2_b
